# baseline (speedup 1.0000x reference)
import functools

import jax
import jax.numpy as jnp
from jax import lax
from jax.experimental import pallas as pl
from jax.experimental.pallas import tpu as pltpu
from jax.experimental.pallas import tpu_sc as plsc

_NC = 2
_NS = 16


_NBUF = 4


def _sc_agg_body(npad, dh, nchunk, chunk, x_hbm, src_hbm, dst_hbm, out_hbm,
                 idx_s, idx_d, rows, zbuf, acc, *sems):
    c = lax.axis_index("c")
    s = lax.axis_index("s")
    rpt = npad // _NS
    zrows = zbuf.shape[0]
    gsem = sems[:_NBUF]
    ssem = sems[_NBUF:]
    x_lin = x_hbm

    def zero_row(i, carry):
        for l in range(dh // 16):
            zbuf[i, pl.ds(l * 16, 16)] = jnp.zeros((16,), jnp.float32)
        return carry
    lax.fori_loop(0, zrows, zero_row, 0)
    for j in range(rpt // zrows):
        pltpu.sync_copy(zbuf, acc.at[pl.ds(s * rpt + j * zrows, zrows)])

    pltpu.sync_copy(src_hbm.at[c, s], idx_s)
    pltpu.sync_copy(dst_hbm.at[s], idx_d)

    def start_gather(k, b):
        pltpu.async_copy(x_lin.at[idx_s.at[k]], rows.at[b], gsem[b])

    def wait_gather(b):
        pltpu.make_async_copy(x_lin.at[pl.ds(0, chunk)],
                              rows.at[b], gsem[b]).wait()

    def start_scatter(k, b):
        pltpu.async_copy(rows.at[b], acc.at[idx_d.at[k]], ssem[b], add=True)

    def wait_scatter(b):
        pltpu.make_async_copy(rows.at[b], acc.at[pl.ds(0, chunk)],
                              ssem[b]).wait()

    for b in range(_NBUF):
        start_gather(b, b)
    plsc.subcore_barrier()

    def outer(k, carry):
        a0 = _NBUF * k
        for b in range(_NBUF):
            wait_gather(b)
            start_scatter(a0 + b, b)
        for b in range(_NBUF):
            wait_scatter(b)
            start_gather(a0 + _NBUF + b, b)
        return carry
    lax.fori_loop(0, nchunk // _NBUF - 1, outer, 0)

    for b in range(_NBUF):
        wait_gather(b)
        start_scatter(nchunk - _NBUF + b, b)
    for b in range(_NBUF):
        wait_scatter(b)
    plsc.subcore_barrier()

    pltpu.sync_copy(acc.at[pl.ds(s * rpt, rpt)],
                    out_hbm.at[pl.ds(s * rpt, rpt), pl.ds(c * dh, dh)])


def _sc_aggregate(x2, src2, dst):
    n2, dh = x2.shape
    n = n2 // 2
    e = dst.shape[0]
    npad = -(-n // (128 * _NS)) * (128 * _NS)
    ept = e // _NS
    chunk = 125
    nchunk = ept // chunk
    src_r = src2.reshape(_NC, _NS, nchunk, chunk)
    dst_r = dst.reshape(_NS, nchunk, chunk)
    mesh = plsc.VectorSubcoreMesh(core_axis_name="c", subcore_axis_name="s")
    return pl.kernel(
        functools.partial(_sc_agg_body, npad, dh, nchunk, chunk),
        out_type=jax.ShapeDtypeStruct((npad, 2 * dh), jnp.float32),
        mesh=mesh,
        scratch_types=[
            pltpu.VMEM((nchunk, chunk), jnp.int32),
            pltpu.VMEM((nchunk, chunk), jnp.int32),
            pltpu.VMEM((_NBUF, chunk, dh), jnp.float32),
            pltpu.VMEM((128, dh), jnp.float32),
            pltpu.VMEM_SHARED((npad, dh), jnp.float32),
        ] + [pltpu.SemaphoreType.DMA] * (2 * _NBUF),
        compiler_params=pltpu.CompilerParams(use_tc_tiling_on_sc=False),
    )(x2, src_r, dst_r)


def _tc_body(nblk, blk, x_ref, a_ref, wl_ref, bl_ref, w1_ref,
             w2_ref, w3_ref, b3_ref, ep_ref, g1_ref, be1_ref, g2_ref,
             be2_ref, o_ref, h1_s, h2_s, s1, q1, s2, q2, co1, co2):
    p = pl.program_id(0)
    i = pl.program_id(1)
    n = nblk * blk

    def bdot(a, b):
        return jnp.dot(a.astype(jnp.bfloat16), b.astype(jnp.bfloat16),
                       preferred_element_type=jnp.float32)

    def stats_accum(v, s_ref, q_ref):
        s = jnp.sum(v, axis=0, keepdims=True)
        q = jnp.sum(v * v, axis=0, keepdims=True)

        @pl.when(i == 0)
        def _():
            s_ref[...] = s
            q_ref[...] = q

        @pl.when(i != 0)
        def _():
            s_ref[...] += s
            q_ref[...] += q

    def bn_coeffs(s_ref, q_ref, g_ref, b_ref, co_ref):
        mean = s_ref[...] / n
        var = q_ref[...] / n - mean * mean
        scale = g_ref[...] / jnp.sqrt(var + 1e-5)
        co_ref[0:1, :] = scale
        co_ref[1:2, :] = b_ref[...] - mean * scale

    @pl.when(p == 0)
    def _():
        xb = x_ref[...]
        t = ep_ref[0, 0] * xb + a_ref[...]
        hh = bdot(xb, wl_ref[...])
        hh = hh + bl_ref[...] + jnp.concatenate([t, t], axis=-1)
        h1 = bdot(hh, w1_ref[...])
        h1_s[pl.ds(i * blk, blk), :] = h1
        stats_accum(h1, s1, q1)

    @pl.when(p == 1)
    def _():
        @pl.when(i == 0)
        def _():
            bn_coeffs(s1, q1, g1_ref, be1_ref, co1)
        h1 = h1_s[pl.ds(i * blk, blk), :]
        a = jnp.maximum(h1 * co1[0:1, :] + co1[1:2, :], 0.0)
        h2 = bdot(a, w2_ref[...])
        h2_s[pl.ds(i * blk, blk), :] = h2
        stats_accum(h2, s2, q2)

    @pl.when(p == 2)
    def _():
        @pl.when(i == 0)
        def _():
            bn_coeffs(s2, q2, g2_ref, be2_ref, co2)
        h2 = h2_s[pl.ds(i * blk, blk), :]
        a = jnp.maximum(h2 * co2[0:1, :] + co2[1:2, :], 0.0)
        o_ref[...] = bdot(a, w3_ref[...]) + b3_ref[...]


def kernel(x, edge_index, lin_W, lin_b, eps, W1, g1, b1, W2, g2, b2, W3, b3):
    n, d = x.shape
    h = lin_W.shape[1]
    dh = d // 2
    src2 = jnp.stack([edge_index[0] * 2, edge_index[0] * 2 + 1])
    agg2 = _sc_aggregate(x.reshape(2 * n, dh), src2, edge_index[1])

    blk = 1000
    nblk = n // blk
    grid = (3, nblk)
    epp1 = (1.0 + eps).reshape(1, 1).astype(jnp.float32)

    def full(shape):
        return pl.BlockSpec(shape, lambda p, i: (0,) * len(shape))

    def rows_spec(cols):
        return pl.BlockSpec((blk, cols),
                            lambda p, i: (jnp.where(p == 0, i, 0), 0))

    out_spec = pl.BlockSpec((blk, d), lambda p, i: (jnp.where(p == 2, i, 0), 0))

    vec_h = full((1, h))
    out = pl.pallas_call(
        functools.partial(_tc_body, nblk, blk),
        grid=grid,
        in_specs=[rows_spec(d), rows_spec(d), full((d, h)),
                  vec_h, full((h, h)), full((h, h)), full((h, d)),
                  full((1, d)), full((1, 1)), vec_h, vec_h, vec_h, vec_h],
        out_specs=out_spec,
        out_shape=jax.ShapeDtypeStruct((n, d), jnp.float32),
        scratch_shapes=[
            pltpu.VMEM((n, h), jnp.float32),
            pltpu.VMEM((n, h), jnp.float32),
            pltpu.VMEM((1, h), jnp.float32),
            pltpu.VMEM((1, h), jnp.float32),
            pltpu.VMEM((1, h), jnp.float32),
            pltpu.VMEM((1, h), jnp.float32),
            pltpu.VMEM((2, h), jnp.float32),
            pltpu.VMEM((2, h), jnp.float32),
        ],
    )(x, agg2, lin_W, lin_b.reshape(1, h), W1, W2, W3,
      b3.reshape(1, d), epp1, g1.reshape(1, h), b1.reshape(1, h),
      g2.reshape(1, h), b2.reshape(1, h))
    return out


# speedup vs baseline: 1.0021x; 1.0021x over previous
import functools

import jax
import jax.numpy as jnp
from jax import lax
from jax.experimental import pallas as pl
from jax.experimental.pallas import tpu as pltpu
from jax.experimental.pallas import tpu_sc as plsc

_NC = 2
_NS = 16


_NBUF = 4


def _sc_agg_body(npad, dh, nchunk, chunk, x_hbm, src_hbm, dst_hbm, out_hbm,
                 idx_s, idx_d, rows, zbuf, acc, *sems):
    c = lax.axis_index("c")
    s = lax.axis_index("s")
    rpt = npad // _NS
    zrows = zbuf.shape[0]
    gsem = sems[:_NBUF]
    ssem = sems[_NBUF:]
    x_lin = x_hbm

    def zero_row(i, carry):
        for l in range(dh // 16):
            zbuf[i, pl.ds(l * 16, 16)] = jnp.zeros((16,), jnp.float32)
        return carry
    lax.fori_loop(0, zrows, zero_row, 0)
    for j in range(rpt // zrows):
        pltpu.sync_copy(zbuf, acc.at[pl.ds(s * rpt + j * zrows, zrows)])

    pltpu.sync_copy(src_hbm.at[c, s], idx_s)
    pltpu.sync_copy(dst_hbm.at[s], idx_d)

    def start_gather(k, b):
        pltpu.async_copy(x_lin.at[idx_s.at[k]], rows.at[b], gsem[b])

    def wait_gather(b):
        pltpu.make_async_copy(x_lin.at[pl.ds(0, chunk)],
                              rows.at[b], gsem[b]).wait()

    def start_scatter(k, b):
        pltpu.async_copy(rows.at[b], acc.at[idx_d.at[k]], ssem[b], add=True)

    def wait_scatter(b):
        pltpu.make_async_copy(rows.at[b], acc.at[pl.ds(0, chunk)],
                              ssem[b]).wait()

    for b in range(_NBUF):
        start_gather(b, b)
    plsc.subcore_barrier()

    def outer(k, carry):
        a0 = _NBUF * k
        for b in range(_NBUF):
            wait_gather(b)
            start_scatter(a0 + b, b)
        for b in range(_NBUF):
            wait_scatter(b)
            start_gather(a0 + _NBUF + b, b)
        return carry
    lax.fori_loop(0, nchunk // _NBUF - 1, outer, 0)

    for b in range(_NBUF):
        wait_gather(b)
        start_scatter(nchunk - _NBUF + b, b)
    for b in range(_NBUF):
        wait_scatter(b)
    plsc.subcore_barrier()

    pltpu.sync_copy(acc.at[pl.ds(s * rpt, rpt)],
                    out_hbm.at[pl.ds(s * rpt, rpt), pl.ds(c * dh, dh)])


def _sc_aggregate(x2, src2, dst):
    n2, dh = x2.shape
    n = n2 // 2
    e = dst.shape[0]
    npad = -(-n // (128 * _NS)) * (128 * _NS)
    ept = e // _NS
    chunk = 125
    nchunk = ept // chunk
    src_r = src2.reshape(_NC, _NS, nchunk, chunk)
    dst_r = dst.reshape(_NS, nchunk, chunk)
    mesh = plsc.VectorSubcoreMesh(core_axis_name="c", subcore_axis_name="s")
    return pl.kernel(
        functools.partial(_sc_agg_body, npad, dh, nchunk, chunk),
        out_type=jax.ShapeDtypeStruct((npad, 2 * dh), jnp.float32),
        mesh=mesh,
        scratch_types=[
            pltpu.VMEM((nchunk, chunk), jnp.int32),
            pltpu.VMEM((nchunk, chunk), jnp.int32),
            pltpu.VMEM((_NBUF, chunk, dh), jnp.float32),
            pltpu.VMEM((128, dh), jnp.float32),
            pltpu.VMEM_SHARED((npad, dh), jnp.float32),
        ] + [pltpu.SemaphoreType.DMA] * (2 * _NBUF),
        compiler_params=pltpu.CompilerParams(use_tc_tiling_on_sc=False),
    )(x2, src_r, dst_r)


def _tc_body(nblk, blk, x_ref, a_ref, wl_ref, bl_ref, w1_ref,
             w2_ref, w3_ref, b3_ref, ep_ref, g1_ref, be1_ref, g2_ref,
             be2_ref, o_ref, h1_s, h2_s, s1, q1, s2, q2, co1, co2):
    p = pl.program_id(0)
    i = pl.program_id(1)
    n = nblk * blk

    def bdot(a, b):
        return jnp.dot(a, b, preferred_element_type=jnp.float32)

    def stats_accum(v, s_ref, q_ref):
        s = jnp.sum(v, axis=0, keepdims=True)
        q = jnp.sum(v * v, axis=0, keepdims=True)

        @pl.when(i == 0)
        def _():
            s_ref[...] = s
            q_ref[...] = q

        @pl.when(i != 0)
        def _():
            s_ref[...] += s
            q_ref[...] += q

    def bn_coeffs(s_ref, q_ref, g_ref, b_ref, co_ref):
        mean = s_ref[...] / n
        var = q_ref[...] / n - mean * mean
        scale = g_ref[...] / jnp.sqrt(var + 1e-5)
        co_ref[0:1, :] = scale
        co_ref[1:2, :] = b_ref[...] - mean * scale

    @pl.when(p == 0)
    def _():
        xb = x_ref[...]
        t = ep_ref[0, 0] * xb + a_ref[...]
        hh = bdot(xb, wl_ref[...])
        hh = hh + bl_ref[...] + jnp.concatenate([t, t], axis=-1)
        h1 = bdot(hh, w1_ref[...])
        h1_s[pl.ds(i * blk, blk), :] = h1
        stats_accum(h1, s1, q1)

    @pl.when(p == 1)
    def _():
        @pl.when(i == 0)
        def _():
            bn_coeffs(s1, q1, g1_ref, be1_ref, co1)
        h1 = h1_s[pl.ds(i * blk, blk), :]
        a = jnp.maximum(h1 * co1[0:1, :] + co1[1:2, :], 0.0)
        h2 = bdot(a, w2_ref[...])
        h2_s[pl.ds(i * blk, blk), :] = h2
        stats_accum(h2, s2, q2)

    @pl.when(p == 2)
    def _():
        @pl.when(i == 0)
        def _():
            bn_coeffs(s2, q2, g2_ref, be2_ref, co2)
        h2 = h2_s[pl.ds(i * blk, blk), :]
        a = jnp.maximum(h2 * co2[0:1, :] + co2[1:2, :], 0.0)
        o_ref[...] = bdot(a, w3_ref[...]) + b3_ref[...]


def kernel(x, edge_index, lin_W, lin_b, eps, W1, g1, b1, W2, g2, b2, W3, b3):
    n, d = x.shape
    h = lin_W.shape[1]
    dh = d // 2
    src2 = jnp.stack([edge_index[0] * 2, edge_index[0] * 2 + 1])
    agg2 = _sc_aggregate(x.reshape(2 * n, dh), src2, edge_index[1])

    blk = 1000
    nblk = n // blk
    grid = (3, nblk)
    epp1 = (1.0 + eps).reshape(1, 1).astype(jnp.float32)

    def full(shape):
        return pl.BlockSpec(shape, lambda p, i: (0,) * len(shape))

    def rows_spec(cols):
        return pl.BlockSpec((blk, cols),
                            lambda p, i: (jnp.where(p == 0, i, 0), 0))

    out_spec = pl.BlockSpec((blk, d), lambda p, i: (jnp.where(p == 2, i, 0), 0))

    vec_h = full((1, h))
    out = pl.pallas_call(
        functools.partial(_tc_body, nblk, blk),
        grid=grid,
        in_specs=[rows_spec(d), rows_spec(d), full((d, h)),
                  vec_h, full((h, h)), full((h, h)), full((h, d)),
                  full((1, d)), full((1, 1)), vec_h, vec_h, vec_h, vec_h],
        out_specs=out_spec,
        out_shape=jax.ShapeDtypeStruct((n, d), jnp.float32),
        scratch_shapes=[
            pltpu.VMEM((n, h), jnp.float32),
            pltpu.VMEM((n, h), jnp.float32),
            pltpu.VMEM((1, h), jnp.float32),
            pltpu.VMEM((1, h), jnp.float32),
            pltpu.VMEM((1, h), jnp.float32),
            pltpu.VMEM((1, h), jnp.float32),
            pltpu.VMEM((2, h), jnp.float32),
            pltpu.VMEM((2, h), jnp.float32),
        ],
    )(x, agg2, lin_W, lin_b.reshape(1, h), W1, W2, W3,
      b3.reshape(1, d), epp1, g1.reshape(1, h), b1.reshape(1, h),
      g2.reshape(1, h), b2.reshape(1, h))
    return out
